# baseline (speedup 1.0000x reference)
import functools

import jax
import jax.numpy as jnp
from jax import lax
from jax.experimental import pallas as pl
from jax.experimental.pallas import tpu as pltpu
from jax.experimental.pallas import tpu_sc as plsc

_NC = 2
_NS = 16
_NW = _NC * _NS
_CH = 128


def _sc_gather(flat_idx, row16, lo16, emb2_flat, emb1_rows):
  FB = flat_idx.shape[0]
  E = emb2_flat.shape[1]
  rows_per_w = FB // _NW
  n_chunks = rows_per_w // _CH
  assert rows_per_w % _CH == 0

  mesh = plsc.VectorSubcoreMesh(core_axis_name="c", subcore_axis_name="s")

  @functools.partial(
      pl.kernel,
      out_type=(
          jax.ShapeDtypeStruct((FB, E), jnp.float32),
          jax.ShapeDtypeStruct((FB,), jnp.float32),
      ),
      mesh=mesh,
      compiler_params=pltpu.CompilerParams(
          use_tc_tiling_on_sc=False, needs_layout_passes=False),
      scratch_types=[
          pltpu.VMEM((_CH,), jnp.int32),
          pltpu.VMEM((_CH,), jnp.int32),
          pltpu.VMEM((_CH,), jnp.int32),
          pltpu.VMEM((_CH, E), jnp.float32),
          pltpu.VMEM((_CH, 16), jnp.float32),
          pltpu.VMEM((_CH,), jnp.float32),
          pltpu.SemaphoreType.DMA,
          pltpu.SemaphoreType.DMA,
      ],
  )
  def sc_k(fidx_hbm, ridx_hbm, lo_hbm, emb2_hbm, e1_hbm, g_out, v_out,
           fidx_v, ridx_v, lo_v, rows_v, vals16_v, vals_v, sem_a, sem_b):
    wid = lax.axis_index("s") * _NC + lax.axis_index("c")
    base = wid * rows_per_w

    def body(c, carry):
      off = base + c * _CH
      pltpu.sync_copy(fidx_hbm.at[pl.ds(off, _CH)], fidx_v)
      pltpu.sync_copy(ridx_hbm.at[pl.ds(off, _CH)], ridx_v)
      pltpu.sync_copy(lo_hbm.at[pl.ds(off, _CH)], lo_v)
      cp_a = pltpu.async_copy(emb2_hbm.at[fidx_v], rows_v, sem_a)
      cp_b = pltpu.async_copy(e1_hbm.at[ridx_v], vals16_v, sem_b)
      cp_a.wait()
      cp_b.wait()

      for t in range(_CH // 16):
        rowids = lax.iota(jnp.int32, 16) + t * 16
        lanes = lo_v[pl.ds(t * 16, 16)]
        vals_v[pl.ds(t * 16, 16)] = plsc.load_gather(
            vals16_v, [rowids, lanes])
      pltpu.sync_copy(rows_v, g_out.at[pl.ds(off, _CH)])
      pltpu.sync_copy(vals_v, v_out.at[pl.ds(off, _CH)])
      return carry

    lax.fori_loop(0, n_chunks, body, 0)

  return sc_k(flat_idx, row16, lo16, emb2_flat, emb1_rows)


def _tc_body(g_ref, v_ref, d_ref, wld_ref, bld_ref, w2_ref, ba1_ref,
             wa2_ref, wf_ref, bf_ref, wl_ref, bl_ref, of_ref, ol_ref):
  Fdim, HB, L = g_ref.shape
  E = L // 2
  i = pl.program_id(0)
  gv = g_ref[...]
  w2 = w2_ref[...]
  ba1 = ba1_ref[...]
  wa2 = wa2_ref[...]

  chunks = [[1]] + [[d, Fdim + 1 - d] for d in range(2, Fdim // 2 + 1)]
  pc = Fdim - 1

  m_l = jnp.full((HB,), -jnp.inf, dtype=jnp.float32)
  m_h = jnp.full((HB,), -jnp.inf, dtype=jnp.float32)
  den_l = jnp.zeros((HB,), dtype=jnp.float32)
  den_h = jnp.zeros((HB,), dtype=jnp.float32)
  num_l = jnp.zeros((HB,), dtype=jnp.float32)
  num_h = jnp.zeros((HB,), dtype=jnp.float32)
  for ds in chunks:
    prods = [(gv[:Fdim - d] * gv[d:]).reshape((Fdim - d) * HB, L)
             for d in ds]
    p2 = prods[0] if len(prods) == 1 else jnp.concatenate(prods, axis=0)
    z = jnp.maximum(
        jnp.dot(p2, w2, preferred_element_type=jnp.float32) + ba1, 0.0)
    z3 = (z * wa2).reshape(pc, HB, L)
    p3 = p2.reshape(pc, HB, L)
    s_l = jnp.sum(z3[:, :, :E], axis=2)
    s_h = jnp.sum(z3[:, :, E:], axis=2)
    rs_l = jnp.sum(p3[:, :, :E], axis=2)
    rs_h = jnp.sum(p3[:, :, E:], axis=2)
    mn_l = jnp.maximum(m_l, jnp.max(s_l, axis=0))
    mn_h = jnp.maximum(m_h, jnp.max(s_h, axis=0))
    e_l = jnp.exp(s_l - mn_l[None, :])
    e_h = jnp.exp(s_h - mn_h[None, :])
    sc_l = jnp.exp(m_l - mn_l)
    sc_h = jnp.exp(m_h - mn_h)
    den_l = den_l * sc_l + jnp.sum(e_l, axis=0)
    den_h = den_h * sc_h + jnp.sum(e_h, axis=0)
    num_l = num_l * sc_l + jnp.sum(e_l * rs_l, axis=0)
    num_h = num_h * sc_h + jnp.sum(e_h * rs_h, axis=0)
    m_l = mn_l
    m_h = mn_h

  afm_l = num_l / den_l
  afm_h = num_h / den_h

  vblk = v_ref[:, pl.ds(i * 2, 2), :]
  first2 = jnp.sum(vblk, axis=0)
  dblk = d_ref[:, pl.ds(i * 2, 2), :]
  wld3 = wld_ref[...].reshape(wld_ref.shape[0], 1, 1)
  ld2 = jnp.sum(dblk * wld3, axis=0) + bld_ref[0]
  logit_l = ld2[0] + first2[0] + afm_l
  logit_h = ld2[1] + first2[1] + afm_h
  of_ref[i * 2, :] = jax.nn.sigmoid(logit_l * wf_ref[0] + bf_ref[0])
  of_ref[i * 2 + 1, :] = jax.nn.sigmoid(logit_h * wf_ref[0] + bf_ref[0])
  ol_ref[i * 2, :] = jax.nn.sigmoid(logit_l * wl_ref[0] + bl_ref[0])
  ol_ref[i * 2 + 1, :] = jax.nn.sigmoid(logit_h * wl_ref[0] + bl_ref[0])


def _tc_afm(g128, v3, dense3, W_ld, b_ld, W2, b_a1, W_a2,
            W_f, b_f, W_l, b_l, interpret=False):
  Fdim, HB_all, L = g128.shape
  B = 2 * HB_all
  D = dense3.shape[0]
  BB = 256
  grid = B // BB
  E = L // 2
  NR = B // L
  smem = pl.BlockSpec(memory_space=pltpu.SMEM)
  ba1_2 = jnp.concatenate([b_a1, b_a1]).reshape(1, L)
  wa2_2 = jnp.concatenate([W_a2.reshape(E), W_a2.reshape(E)]).reshape(1, L)
  return pl.pallas_call(
      _tc_body,
      grid=(grid,),
      in_specs=[
          pl.BlockSpec((Fdim, BB // 2, L), lambda i: (0, i, 0)),
          pl.BlockSpec((Fdim, NR, L), lambda i: (0, 0, 0)),
          pl.BlockSpec((D, NR, L), lambda i: (0, 0, 0)),
          pl.BlockSpec((D, 1), lambda i: (0, 0)),
          smem,
          pl.BlockSpec((L, L), lambda i: (0, 0)),
          pl.BlockSpec((1, L), lambda i: (0, 0)),
          pl.BlockSpec((1, L), lambda i: (0, 0)),
          smem,
          smem,
          smem,
          smem,
      ],
      out_specs=[
          pl.BlockSpec((NR, L), lambda i: (0, 0)),
          pl.BlockSpec((NR, L), lambda i: (0, 0)),
      ],
      out_shape=[jax.ShapeDtypeStruct((NR, L), jnp.float32)] * 2,
      interpret=interpret,
  )(g128, v3, dense3, W_ld, b_ld, W2, ba1_2, wa2_2,
    W_f.reshape(1), b_f, W_l.reshape(1), b_l)


def kernel(sparse_inputs, dense_inputs, emb1, emb2, W_ld, b_ld, W_a1, b_a1,
           W_a2, b_a2, W_f, b_f, W_l, b_l):
  F, V, E = emb2.shape
  B = sparse_inputs.shape[0]
  D = dense_inputs.shape[1]
  L = 2 * E

  idxT = sparse_inputs.T.astype(jnp.int32)
  flat2 = idxT + (jnp.arange(F, dtype=jnp.int32) * V)[:, None]
  flat_idx = flat2.reshape(-1)
  flat_g = flat2.reshape(F, B // 256, 2, 128).transpose(0, 1, 3, 2).reshape(-1)
  row16 = flat_idx // 16
  lo16 = flat_idx % 16
  emb2_flat = emb2.reshape(F * V, E)
  emb1_rows = emb1.reshape(F * V // 16, 16)

  g_flat, v_flat = _sc_gather(flat_g, row16, lo16, emb2_flat, emb1_rows)
  g128 = g_flat.reshape(F, B // 2, L)
  v3 = v_flat.reshape(F, B // L, L)

  zero = jnp.zeros((E, E), jnp.float32)
  W2 = jnp.concatenate(
      [jnp.concatenate([W_a1, zero], axis=1),
       jnp.concatenate([zero, W_a1], axis=1)], axis=0)

  dense3 = dense_inputs.T.reshape(D, B // L, L)
  fo, lo = _tc_afm(g128, v3, dense3, W_ld, b_ld, W2, b_a1, W_a2,
                   W_f, b_f, W_l, b_l)
  return fo.reshape(B, 1), lo.reshape(B, 1)


# speedup vs baseline: 1.1490x; 1.1490x over previous
import functools

import jax
import jax.numpy as jnp
from jax import lax
from jax.experimental import pallas as pl
from jax.experimental.pallas import tpu as pltpu
from jax.experimental.pallas import tpu_sc as plsc

_NC = 2
_NS = 16
_NW = _NC * _NS
_CH = 128


def _sc_gather(flat_idx, row16, lo16, emb2_flat, emb1_rows):
  FB = flat_idx.shape[0]
  E = emb2_flat.shape[1]
  rows_per_w = FB // _NW
  n_chunks = rows_per_w // _CH
  assert rows_per_w % _CH == 0

  mesh = plsc.VectorSubcoreMesh(core_axis_name="c", subcore_axis_name="s")

  @functools.partial(
      pl.kernel,
      out_type=(
          jax.ShapeDtypeStruct((FB, E), jnp.float32),
          jax.ShapeDtypeStruct((FB,), jnp.float32),
      ),
      mesh=mesh,
      compiler_params=pltpu.CompilerParams(
          use_tc_tiling_on_sc=False, needs_layout_passes=False),
      scratch_types=[
          pltpu.VMEM((_CH,), jnp.int32),
          pltpu.VMEM((_CH,), jnp.int32),
          pltpu.VMEM((_CH,), jnp.int32),
          pltpu.VMEM((_CH, E), jnp.float32),
          pltpu.VMEM((_CH, 16), jnp.float32),
          pltpu.VMEM((_CH,), jnp.float32),
          pltpu.SemaphoreType.DMA,
          pltpu.SemaphoreType.DMA,
      ],
  )
  def sc_k(fidx_hbm, ridx_hbm, lo_hbm, emb2_hbm, e1_hbm, g_out, v_out,
           fidx_v, ridx_v, lo_v, rows_v, vals16_v, vals_v, sem_a, sem_b):
    wid = lax.axis_index("s") * _NC + lax.axis_index("c")
    base = wid * rows_per_w

    def body(c, carry):
      off = base + c * _CH
      pltpu.sync_copy(fidx_hbm.at[pl.ds(off, _CH)], fidx_v)
      pltpu.sync_copy(ridx_hbm.at[pl.ds(off, _CH)], ridx_v)
      pltpu.sync_copy(lo_hbm.at[pl.ds(off, _CH)], lo_v)
      cp_a = pltpu.async_copy(emb2_hbm.at[fidx_v], rows_v, sem_a)
      cp_b = pltpu.async_copy(e1_hbm.at[ridx_v], vals16_v, sem_b)
      cp_a.wait()
      cp_b.wait()
      for t in range(_CH // 16):
        rowids = lax.iota(jnp.int32, 16) + t * 16
        lanes = lo_v[pl.ds(t * 16, 16)]
        vals_v[pl.ds(t * 16, 16)] = plsc.load_gather(
            vals16_v, [rowids, lanes])
      pltpu.sync_copy(rows_v, g_out.at[pl.ds(off, _CH)])
      pltpu.sync_copy(vals_v, v_out.at[pl.ds(off, _CH)])
      return carry

    lax.fori_loop(0, n_chunks, body, 0)

  return sc_k(flat_idx, row16, lo16, emb2_flat, emb1_rows)


def _tc_body(g_ref, v_ref, d_ref, wld_ref, bld_ref, wa1_ref, ba1_ref,
             wa2_ref, wf_ref, bf_ref, wl_ref, bl_ref, of_ref, ol_ref):
  Fdim, BB, E = g_ref.shape
  L = v_ref.shape[2]
  i = pl.program_id(0)
  gv = g_ref[...]
  wa1 = wa1_ref[...]
  ba1 = ba1_ref[...]
  wa2 = wa2_ref[...]

  chunks = [[1]] + [[d, Fdim + 1 - d] for d in range(2, Fdim // 2 + 1)]
  pc = Fdim - 1

  m = jnp.full((BB,), -jnp.inf, dtype=jnp.float32)
  den = jnp.zeros((BB,), dtype=jnp.float32)
  num = jnp.zeros((BB,), dtype=jnp.float32)
  for ds in chunks:
    prods = [(gv[:Fdim - d] * gv[d:]).reshape((Fdim - d) * BB, E)
             for d in ds]
    p2 = prods[0] if len(prods) == 1 else jnp.concatenate(prods, axis=0)
    z = jnp.maximum(
        jnp.dot(p2, wa1, preferred_element_type=jnp.float32) + ba1, 0.0)
    z3 = (z * wa2).reshape(pc, BB, E)
    p3 = p2.reshape(pc, BB, E)
    s_d = jnp.sum(z3, axis=2)
    rs_d = jnp.sum(p3, axis=2)
    m_new = jnp.maximum(m, jnp.max(s_d, axis=0))
    scale = jnp.exp(m - m_new)
    e_d = jnp.exp(s_d - m_new[None, :])
    den = den * scale + jnp.sum(e_d, axis=0)
    num = num * scale + jnp.sum(e_d * rs_d, axis=0)
    m = m_new

  afm = num / den
  nb = BB // L
  vblk = v_ref[:, pl.ds(i * nb, nb), :]
  first2 = jnp.sum(vblk, axis=0)
  first = jnp.concatenate([first2[k] for k in range(nb)], axis=0)
  ld = jnp.sum(d_ref[...] * wld_ref[...], axis=0) + bld_ref[0]
  logits = ld + first + afm
  of_ref[...] = jax.nn.sigmoid(logits * wf_ref[0] + bf_ref[0])
  ol_ref[...] = jax.nn.sigmoid(logits * wl_ref[0] + bl_ref[0])


def _tc_afm(g, v3, dense_T, W_ld, b_ld, W_a1, b_a1, W_a2,
            W_f, b_f, W_l, b_l, interpret=False):
  Fdim, B, E = g.shape
  D = dense_T.shape[0]
  L = v3.shape[2]
  BB = 256
  grid = B // BB
  NR = B // L
  smem = pl.BlockSpec(memory_space=pltpu.SMEM)
  return pl.pallas_call(
      _tc_body,
      grid=(grid,),
      in_specs=[
          pl.BlockSpec((Fdim, BB, E), lambda i: (0, i, 0)),
          pl.BlockSpec((Fdim, NR, L), lambda i: (0, 0, 0)),
          pl.BlockSpec((D, BB), lambda i: (0, i)),
          pl.BlockSpec((D, 1), lambda i: (0, 0)),
          smem,
          pl.BlockSpec((E, E), lambda i: (0, 0)),
          pl.BlockSpec((1, E), lambda i: (0, 0)),
          pl.BlockSpec((1, E), lambda i: (0, 0)),
          smem,
          smem,
          smem,
          smem,
      ],
      out_specs=[
          pl.BlockSpec((BB,), lambda i: (i,)),
          pl.BlockSpec((BB,), lambda i: (i,)),
      ],
      out_shape=[jax.ShapeDtypeStruct((B,), jnp.float32)] * 2,
      interpret=interpret,
  )(g, v3, dense_T, W_ld, b_ld, W_a1, b_a1.reshape(1, E),
    W_a2.reshape(1, E), W_f.reshape(1), b_f, W_l.reshape(1), b_l)


def kernel(sparse_inputs, dense_inputs, emb1, emb2, W_ld, b_ld, W_a1, b_a1,
           W_a2, b_a2, W_f, b_f, W_l, b_l):
  F, V, E = emb2.shape
  B = sparse_inputs.shape[0]
  L = 128

  idxT = sparse_inputs.T.astype(jnp.int32)
  flat_idx = (idxT + (jnp.arange(F, dtype=jnp.int32) * V)[:, None]).reshape(-1)
  row16 = flat_idx // 16
  lo16 = flat_idx % 16
  emb2_flat = emb2.reshape(F * V, E)
  emb1_rows = emb1.reshape(F * V // 16, 16)

  g_flat, v_flat = _sc_gather(flat_idx, row16, lo16, emb2_flat, emb1_rows)
  g = g_flat.reshape(F, B, E)
  v3 = v_flat.reshape(F, B // L, L)

  fo, lo = _tc_afm(g, v3, dense_inputs.T, W_ld, b_ld, W_a1, b_a1, W_a2,
                   W_f, b_f, W_l, b_l)
  return fo.reshape(B, 1), lo.reshape(B, 1)
